# initial kernel scaffold (unmeasured)
import jax
import jax.numpy as jnp
from jax import lax
from jax.experimental import pallas as pl
from jax.experimental.pallas import tpu as pltpu

N_DEV = 4
E_PER = 4
CAPACITY = 204.0


def kernel(x, router_W, route_idx, expert_W):
    del router_W
    m_tok, d_model = x.shape
    _, _, d_ff = expert_W.shape
    n_exp = N_DEV * E_PER

    def body(x_ref, ridx_ref, ew_ref, out_ref,
             cnt_all, comm,
             cnt_send_sems, cnt_recv_sems, w_send_sems, w_recv_sems):
        p = lax.axis_index("i")
        right = lax.rem(p + 1, N_DEV)

        r = ridx_ref[:, :]
        e_iota = lax.broadcasted_iota(jnp.int32, (m_tok, n_exp), 1)
        onehot = (r == e_iota).astype(jnp.float32)
        row = lax.broadcasted_iota(jnp.int32, (m_tok, m_tok), 0)
        col = lax.broadcasted_iota(jnp.int32, (m_tok, m_tok), 1)
        tril = (row > col).astype(jnp.float32)
        excl = jnp.dot(tril, onehot, preferred_element_type=jnp.float32)
        counts_mine = jnp.sum(onehot, axis=0, keepdims=True)
        cnt_all[0, :, :] = counts_mine

        barrier_sem = pltpu.get_barrier_semaphore()
        for k in range(1, N_DEV):
            pl.semaphore_signal(
                barrier_sem, inc=1,
                device_id=(lax.rem(p + k, N_DEV),),
                device_id_type=pl.DeviceIdType.MESH,
            )
        pl.semaphore_wait(barrier_sem, N_DEV - 1)

        cnt_rdmas = []
        for k in range(1, N_DEV):
            rd = pltpu.make_async_remote_copy(
                src_ref=cnt_all.at[0],
                dst_ref=cnt_all.at[k],
                send_sem=cnt_send_sems.at[k - 1],
                recv_sem=cnt_recv_sems.at[k - 1],
                device_id=(lax.rem(p + k, N_DEV),),
                device_id_type=pl.DeviceIdType.MESH,
            )
            rd.start()
            cnt_rdmas.append(rd)
        for rd in cnt_rdmas:
            rd.wait()

        offs = jnp.zeros((1, n_exp), jnp.float32)
        for k in range(1, N_DEV):
            origin = lax.rem(p - k + N_DEV, N_DEV)
            offs = offs + jnp.where(origin < p, cnt_all[k, :, :], 0.0)
        rank_full = excl + offs
        rank_tok = jnp.sum(rank_full * onehot, axis=1, keepdims=True)
        kept = rank_tok < CAPACITY

        xv = x_ref[:, :]

        def block_out(w_ref, origin):
            acc = jnp.zeros((m_tok, d_ff), jnp.float32)
            for j in range(E_PER):
                e = E_PER * origin + j
                m = jnp.where((r == e) & kept, 1.0, 0.0)
                acc = acc + jnp.dot(
                    xv * m, w_ref[j, :, :],
                    preferred_element_type=jnp.float32,
                )
            return acc

        out_ref[:, :] = block_out(ew_ref, p)

        for h in range(N_DEV - 1):
            src = ew_ref if h == 0 else comm.at[h - 1]
            rd = pltpu.make_async_remote_copy(
                src_ref=src,
                dst_ref=comm.at[h],
                send_sem=w_send_sems.at[h],
                recv_sem=w_recv_sems.at[h],
                device_id=(right,),
                device_id_type=pl.DeviceIdType.MESH,
            )
            rd.start()
            rd.wait()
            origin = lax.rem(p - h - 1 + N_DEV, N_DEV)
            out_ref[:, :] = out_ref[:, :] + block_out(comm.at[h], origin)

    return pl.pallas_call(
        body,
        out_shape=jax.ShapeDtypeStruct((m_tok, d_ff), jnp.float32),
        in_specs=[
            pl.BlockSpec(memory_space=pltpu.VMEM),
            pl.BlockSpec(memory_space=pltpu.VMEM),
            pl.BlockSpec(memory_space=pltpu.VMEM),
        ],
        out_specs=pl.BlockSpec(memory_space=pltpu.VMEM),
        scratch_shapes=[
            pltpu.VMEM((N_DEV, 1, 16), jnp.float32),
            pltpu.VMEM((N_DEV - 1, E_PER, d_model, d_ff), jnp.float32),
            pltpu.SemaphoreType.DMA((N_DEV - 1,)),
            pltpu.SemaphoreType.DMA((N_DEV - 1,)),
            pltpu.SemaphoreType.DMA((N_DEV - 1,)),
            pltpu.SemaphoreType.DMA((N_DEV - 1,)),
        ],
        compiler_params=pltpu.CompilerParams(collective_id=0),
    )(x, route_idx, expert_W)


# baseline (device time: 323034 ns/iter reference)
import jax
import jax.numpy as jnp
from jax import lax
from jax.experimental import pallas as pl
from jax.experimental.pallas import tpu as pltpu

N_DEV = 4
E_PER = 4
CAPACITY = 204.0


def kernel(x, router_W, route_idx, expert_W):
    del router_W
    m_tok, d_model = x.shape
    _, _, d_ff = expert_W.shape
    n_exp = N_DEV * E_PER

    def body(x_ref, ridx_ref, ew_ref, out_ref,
             cnt_all, comm,
             cnt_send_sems, cnt_recv_sems, w_send_sems, w_recv_sems):
        p = lax.axis_index("i")
        right = lax.rem(p + 1, N_DEV)

        r = ridx_ref[:, :]
        e_iota = lax.broadcasted_iota(jnp.int32, (m_tok, n_exp), 1)
        onehot = (r == e_iota).astype(jnp.float32)
        row = lax.broadcasted_iota(jnp.int32, (m_tok, m_tok), 0)
        col = lax.broadcasted_iota(jnp.int32, (m_tok, m_tok), 1)
        tril = (row > col).astype(jnp.float32)
        excl = jnp.dot(tril, onehot, preferred_element_type=jnp.float32)
        counts_mine = jnp.sum(onehot, axis=0, keepdims=True)
        cnt_all[0, :, :] = counts_mine

        barrier_sem = pltpu.get_barrier_semaphore()
        for k in range(1, N_DEV):
            pl.semaphore_signal(
                barrier_sem, inc=1,
                device_id=(lax.rem(p + k, N_DEV),),
                device_id_type=pl.DeviceIdType.MESH,
            )
        pl.semaphore_wait(barrier_sem, N_DEV - 1)

        cnt_rdmas = []
        for k in range(1, N_DEV):
            rd = pltpu.make_async_remote_copy(
                src_ref=cnt_all.at[0],
                dst_ref=cnt_all.at[k],
                send_sem=cnt_send_sems.at[k - 1],
                recv_sem=cnt_recv_sems.at[k - 1],
                device_id=(lax.rem(p + k, N_DEV),),
                device_id_type=pl.DeviceIdType.MESH,
            )
            rd.start()
            cnt_rdmas.append(rd)
        for rd in cnt_rdmas:
            rd.wait()

        offs = jnp.zeros((1, n_exp), jnp.float32)
        for k in range(1, N_DEV):
            origin = lax.rem(p - k + N_DEV, N_DEV)
            offs = offs + jnp.where(origin < p, cnt_all[k, :, :], 0.0)
        rank_full = excl + offs
        rank_tok = jnp.sum(rank_full * onehot, axis=1, keepdims=True)
        kept = rank_tok < CAPACITY

        xv = x_ref[:, :]

        def block_out(w_ref, origin):
            acc = jnp.zeros((m_tok, d_ff), jnp.float32)
            for j in range(E_PER):
                e = E_PER * origin + j
                m = jnp.where((r == e) & kept, 1.0, 0.0)
                acc = acc + jnp.dot(
                    xv * m, w_ref[j, :, :],
                    preferred_element_type=jnp.float32,
                )
            return acc

        out_ref[:, :] = block_out(ew_ref, p)

        for h in range(N_DEV - 1):
            src = ew_ref if h == 0 else comm.at[h - 1]
            rd = pltpu.make_async_remote_copy(
                src_ref=src,
                dst_ref=comm.at[h],
                send_sem=w_send_sems.at[h],
                recv_sem=w_recv_sems.at[h],
                device_id=(right,),
                device_id_type=pl.DeviceIdType.MESH,
            )
            rd.start()
            rd.wait()
            origin = lax.rem(p - h - 1 + N_DEV, N_DEV)
            out_ref[:, :] = out_ref[:, :] + block_out(comm.at[h], origin)

    return pl.pallas_call(
        body,
        out_shape=jax.ShapeDtypeStruct((m_tok, d_ff), jnp.float32),
        in_specs=[
            pl.BlockSpec(memory_space=pltpu.VMEM),
            pl.BlockSpec(memory_space=pltpu.VMEM),
            pl.BlockSpec(memory_space=pltpu.VMEM),
        ],
        out_specs=pl.BlockSpec(memory_space=pltpu.VMEM),
        scratch_shapes=[
            pltpu.VMEM((N_DEV, 1, 16), jnp.float32),
            pltpu.VMEM((N_DEV - 1, E_PER, d_model, d_ff), jnp.float32),
            pltpu.SemaphoreType.DMA((N_DEV - 1,)),
            pltpu.SemaphoreType.DMA((N_DEV - 1,)),
            pltpu.SemaphoreType.DMA((N_DEV - 1,)),
            pltpu.SemaphoreType.DMA((N_DEV - 1,)),
        ],
        compiler_params=pltpu.CompilerParams(
            collective_id=0,
            vmem_limit_bytes=64 * 1024 * 1024,
        ),
    )(x, route_idx, expert_W)


# device time: 173961 ns/iter; 1.8569x vs baseline; 1.8569x over previous
import jax
import jax.numpy as jnp
from jax import lax
from jax.experimental import pallas as pl
from jax.experimental.pallas import tpu as pltpu

N_DEV = 4
E_PER = 4
CAPACITY = 204.0


def kernel(x, router_W, route_idx, expert_W):
    del router_W
    m_tok, d_model = x.shape
    _, _, d_ff = expert_W.shape
    n_exp = N_DEV * E_PER

    def body(x_ref, ridx_ref, ew_ref, out_ref,
             cnt_all, myb, comm,
             cnt_send_sems, cnt_recv_sems, w_send_sems, w_recv_sems):
        p = lax.axis_index("i")
        right = lax.rem(p + 1, N_DEV)

        r = ridx_ref[:, :]
        e_iota = lax.broadcasted_iota(jnp.int32, (m_tok, n_exp), 1)
        onehot = (r == e_iota).astype(jnp.float32)
        row = lax.broadcasted_iota(jnp.int32, (m_tok, m_tok), 0)
        col = lax.broadcasted_iota(jnp.int32, (m_tok, m_tok), 1)
        tril = (row > col).astype(jnp.float32)
        excl = jnp.dot(tril, onehot, preferred_element_type=jnp.float32)
        counts_mine = jnp.sum(onehot, axis=0, keepdims=True)
        cnt_all[0, :, :] = counts_mine

        myb[:, :, :] = ew_ref[:, :, :].astype(jnp.bfloat16)
        xb = x_ref[:, :].astype(jnp.bfloat16)

        barrier_sem = pltpu.get_barrier_semaphore()
        for k in range(1, N_DEV):
            pl.semaphore_signal(
                barrier_sem, inc=1,
                device_id=(lax.rem(p + k, N_DEV),),
                device_id_type=pl.DeviceIdType.MESH,
            )
        pl.semaphore_wait(barrier_sem, N_DEV - 1)

        def hop_rdma(h):
            return pltpu.make_async_remote_copy(
                src_ref=myb if h == 0 else comm.at[h - 1],
                dst_ref=comm.at[h],
                send_sem=w_send_sems.at[h],
                recv_sem=w_recv_sems.at[h],
                device_id=(right,),
                device_id_type=pl.DeviceIdType.MESH,
            )

        w_rdmas = [hop_rdma(0)]
        w_rdmas[0].start()

        cnt_rdmas = []
        for k in range(1, N_DEV):
            rd = pltpu.make_async_remote_copy(
                src_ref=cnt_all.at[0],
                dst_ref=cnt_all.at[k],
                send_sem=cnt_send_sems.at[k - 1],
                recv_sem=cnt_recv_sems.at[k - 1],
                device_id=(lax.rem(p + k, N_DEV),),
                device_id_type=pl.DeviceIdType.MESH,
            )
            rd.start()
            cnt_rdmas.append(rd)
        for rd in cnt_rdmas:
            rd.wait()

        offs = jnp.zeros((1, n_exp), jnp.float32)
        for k in range(1, N_DEV):
            origin = lax.rem(p - k + N_DEV, N_DEV)
            offs = offs + jnp.where(origin < p, cnt_all[k, :, :], 0.0)
        rank_full = excl + offs
        rank_tok = jnp.sum(rank_full * onehot, axis=1, keepdims=True)
        kept = rank_tok < CAPACITY

        def block_out(w_ref, origin):
            acc = jnp.zeros((m_tok, d_ff), jnp.float32)
            for j in range(E_PER):
                e = E_PER * origin + j
                m = jnp.where((r == e) & kept, 1.0, 0.0)
                acc = acc + jnp.dot(
                    xb * m.astype(jnp.bfloat16), w_ref[j, :, :],
                    preferred_element_type=jnp.float32,
                )
            return acc

        out_ref[:, :] = block_out(myb, p)
        for h in range(N_DEV - 1):
            w_rdmas[h].wait_recv()
            if h + 1 < N_DEV - 1:
                w_rdmas.append(hop_rdma(h + 1))
                w_rdmas[h + 1].start()
            origin = lax.rem(p - h - 1 + N_DEV, N_DEV)
            out_ref[:, :] = out_ref[:, :] + block_out(comm.at[h], origin)
        for rd in w_rdmas:
            rd.wait_send()

    return pl.pallas_call(
        body,
        out_shape=jax.ShapeDtypeStruct((m_tok, d_ff), jnp.float32),
        in_specs=[
            pl.BlockSpec(memory_space=pltpu.VMEM),
            pl.BlockSpec(memory_space=pltpu.VMEM),
            pl.BlockSpec(memory_space=pltpu.VMEM),
        ],
        out_specs=pl.BlockSpec(memory_space=pltpu.VMEM),
        scratch_shapes=[
            pltpu.VMEM((N_DEV, 1, 16), jnp.float32),
            pltpu.VMEM((E_PER, d_model, d_ff), jnp.bfloat16),
            pltpu.VMEM((N_DEV - 1, E_PER, d_model, d_ff), jnp.bfloat16),
            pltpu.SemaphoreType.DMA((N_DEV - 1,)),
            pltpu.SemaphoreType.DMA((N_DEV - 1,)),
            pltpu.SemaphoreType.DMA((N_DEV - 1,)),
            pltpu.SemaphoreType.DMA((N_DEV - 1,)),
        ],
        compiler_params=pltpu.CompilerParams(
            collective_id=0,
            vmem_limit_bytes=64 * 1024 * 1024,
        ),
    )(x, route_idx, expert_W)


# device time: 105333 ns/iter; 3.0668x vs baseline; 1.6515x over previous
import jax
import jax.numpy as jnp
from jax import lax
from jax.experimental import pallas as pl
from jax.experimental.pallas import tpu as pltpu

N_DEV = 4
E_PER = 4
CAPACITY = 204.0


def kernel(x, router_W, route_idx, expert_W):
    del router_W
    m_tok, d_model = x.shape
    _, _, d_ff = expert_W.shape
    n_exp = N_DEV * E_PER

    def body(x_ref, ridx_ref, ew_ref, out_ref,
             cnt_all, myb, bL, bR, bD,
             cnt_send_sems, cnt_recv_sems, w_send_sems, w_recv_sems):
        p = lax.axis_index("i")
        right = lax.rem(p + 1, N_DEV)
        left = lax.rem(p + N_DEV - 1, N_DEV)

        r = ridx_ref[:, :]
        e_iota = lax.broadcasted_iota(jnp.int32, (m_tok, n_exp), 1)
        onehot = (r == e_iota).astype(jnp.float32)
        row = lax.broadcasted_iota(jnp.int32, (m_tok, m_tok), 0)
        col = lax.broadcasted_iota(jnp.int32, (m_tok, m_tok), 1)
        tril = (row > col).astype(jnp.float32)
        excl = jnp.dot(tril, onehot, preferred_element_type=jnp.float32)
        counts_mine = jnp.sum(onehot, axis=0, keepdims=True)
        cnt_all[0, :, :] = counts_mine

        myb[:, :, :] = ew_ref[:, :, :].astype(jnp.bfloat16)
        xb = x_ref[:, :].astype(jnp.bfloat16)

        barrier_sem = pltpu.get_barrier_semaphore()
        for k in range(1, N_DEV):
            pl.semaphore_signal(
                barrier_sem, inc=1,
                device_id=(lax.rem(p + k, N_DEV),),
                device_id_type=pl.DeviceIdType.MESH,
            )
        pl.semaphore_wait(barrier_sem, N_DEV - 1)

        def wcopy(src, dst, sem_idx, target):
            return pltpu.make_async_remote_copy(
                src_ref=src, dst_ref=dst,
                send_sem=w_send_sems.at[sem_idx],
                recv_sem=w_recv_sems.at[sem_idx],
                device_id=(target,),
                device_id_type=pl.DeviceIdType.MESH,
            )

        h1r = wcopy(myb, bL, 0, right)
        h1l = wcopy(myb, bR, 1, left)
        h1r.start()
        h1l.start()

        cnt_rdmas = []
        for k in range(1, N_DEV):
            rd = pltpu.make_async_remote_copy(
                src_ref=cnt_all.at[0],
                dst_ref=cnt_all.at[k],
                send_sem=cnt_send_sems.at[k - 1],
                recv_sem=cnt_recv_sems.at[k - 1],
                device_id=(lax.rem(p + k, N_DEV),),
                device_id_type=pl.DeviceIdType.MESH,
            )
            rd.start()
            cnt_rdmas.append(rd)
        for rd in cnt_rdmas:
            rd.wait()

        offs = jnp.zeros((1, n_exp), jnp.float32)
        for k in range(1, N_DEV):
            origin = lax.rem(p - k + N_DEV, N_DEV)
            offs = offs + jnp.where(origin < p, cnt_all[k, :, :], 0.0)
        rank_full = excl + offs
        rank_tok = jnp.sum(rank_full * onehot, axis=1, keepdims=True)
        kept = rank_tok < CAPACITY

        def block_out(w_ref, origin):
            acc = jnp.zeros((m_tok, d_ff), jnp.float32)
            for j in range(E_PER):
                e = E_PER * origin + j
                m = jnp.where((r == e) & kept, 1.0, 0.0)
                acc = acc + jnp.dot(
                    xb * m.astype(jnp.bfloat16), w_ref[j, :, :],
                    preferred_element_type=jnp.float32,
                )
            return acc

        out_ref[:, :] = block_out(myb, p)

        h1r.wait_recv()
        h1l.wait_recv()
        h2r = wcopy(bL.at[pl.ds(0, E_PER // 2)], bD.at[pl.ds(0, E_PER // 2)],
                    2, right)
        h2l = wcopy(bR.at[pl.ds(E_PER // 2, E_PER // 2)],
                    bD.at[pl.ds(E_PER // 2, E_PER // 2)], 3, left)
        h2r.start()
        h2l.start()

        out_ref[:, :] = out_ref[:, :] + block_out(bL, left)
        out_ref[:, :] = out_ref[:, :] + block_out(bR, right)

        h2r.wait_recv()
        h2l.wait_recv()
        out_ref[:, :] = out_ref[:, :] + block_out(bD, lax.rem(p + 2, N_DEV))

        for rd in (h1r, h1l, h2r, h2l):
            rd.wait_send()

    return pl.pallas_call(
        body,
        out_shape=jax.ShapeDtypeStruct((m_tok, d_ff), jnp.float32),
        in_specs=[
            pl.BlockSpec(memory_space=pltpu.VMEM),
            pl.BlockSpec(memory_space=pltpu.VMEM),
            pl.BlockSpec(memory_space=pltpu.VMEM),
        ],
        out_specs=pl.BlockSpec(memory_space=pltpu.VMEM),
        scratch_shapes=[
            pltpu.VMEM((N_DEV, 1, 16), jnp.float32),
            pltpu.VMEM((E_PER, d_model, d_ff), jnp.bfloat16),
            pltpu.VMEM((E_PER, d_model, d_ff), jnp.bfloat16),
            pltpu.VMEM((E_PER, d_model, d_ff), jnp.bfloat16),
            pltpu.VMEM((E_PER, d_model, d_ff), jnp.bfloat16),
            pltpu.SemaphoreType.DMA((N_DEV - 1,)),
            pltpu.SemaphoreType.DMA((N_DEV - 1,)),
            pltpu.SemaphoreType.DMA((4,)),
            pltpu.SemaphoreType.DMA((4,)),
        ],
        compiler_params=pltpu.CompilerParams(
            collective_id=0,
            vmem_limit_bytes=64 * 1024 * 1024,
        ),
    )(x, route_idx, expert_W)


# device time: 73019 ns/iter; 4.4240x vs baseline; 1.4425x over previous
import jax
import jax.numpy as jnp
from jax import lax
from jax.experimental import pallas as pl
from jax.experimental.pallas import tpu as pltpu

N_DEV = 4
E_PER = 4
CAPACITY = 204.0


def kernel(x, router_W, route_idx, expert_W):
    del router_W
    m_tok, d_model = x.shape
    _, _, d_ff = expert_W.shape
    n_exp = N_DEV * E_PER

    def body(x_ref, ridx_ref, ew_ref, out_ref,
             cnt_all, myb, bL, bR, bD,
             cnt_send_sems, cnt_recv_sems, w_send_sems, w_recv_sems):
        p = lax.axis_index("i")
        right = lax.rem(p + 1, N_DEV)
        left = lax.rem(p + N_DEV - 1, N_DEV)

        r = ridx_ref[:, :]
        e_iota = lax.broadcasted_iota(jnp.int32, (m_tok, n_exp), 1)
        onehot = (r == e_iota).astype(jnp.float32)
        row = lax.broadcasted_iota(jnp.int32, (m_tok, m_tok), 0)
        col = lax.broadcasted_iota(jnp.int32, (m_tok, m_tok), 1)
        tril = (row > col).astype(jnp.float32)
        excl = jnp.dot(tril, onehot, preferred_element_type=jnp.float32)
        counts_mine = jnp.sum(onehot, axis=0, keepdims=True)
        cnt_all[0, :, 0:n_exp] = counts_mine

        j_iota = lax.broadcasted_iota(jnp.int32, (1, E_PER), 1)
        srow = jnp.zeros((1, E_PER), jnp.float32)
        for j in range(E_PER):
            w = ew_ref[j, :, :]
            s = jnp.max(jnp.abs(w)) / 127.0
            myb[j, :, :] = jnp.round(w / s).astype(jnp.int8)
            srow = srow + jnp.where(j_iota == j, s, 0.0)
        cnt_all[0, :, n_exp:] = srow
        xb = x_ref[:, :].astype(jnp.bfloat16)

        barrier_sem = pltpu.get_barrier_semaphore()
        for k in range(1, N_DEV):
            pl.semaphore_signal(
                barrier_sem, inc=1,
                device_id=(lax.rem(p + k, N_DEV),),
                device_id_type=pl.DeviceIdType.MESH,
            )
        pl.semaphore_wait(barrier_sem, N_DEV - 1)

        def wcopy(src, dst, sem_idx, target):
            return pltpu.make_async_remote_copy(
                src_ref=src, dst_ref=dst,
                send_sem=w_send_sems.at[sem_idx],
                recv_sem=w_recv_sems.at[sem_idx],
                device_id=(target,),
                device_id_type=pl.DeviceIdType.MESH,
            )

        h1r = wcopy(myb, bL, 0, right)
        h1l = wcopy(myb, bR, 1, left)
        h1r.start()
        h1l.start()

        cnt_rdmas = []
        for k in range(1, N_DEV):
            rd = pltpu.make_async_remote_copy(
                src_ref=cnt_all.at[0],
                dst_ref=cnt_all.at[k],
                send_sem=cnt_send_sems.at[k - 1],
                recv_sem=cnt_recv_sems.at[k - 1],
                device_id=(lax.rem(p + k, N_DEV),),
                device_id_type=pl.DeviceIdType.MESH,
            )
            rd.start()
            cnt_rdmas.append(rd)
        for rd in cnt_rdmas:
            rd.wait()

        offs = jnp.zeros((1, n_exp), jnp.float32)
        for k in range(1, N_DEV):
            origin = lax.rem(p - k + N_DEV, N_DEV)
            offs = offs + jnp.where(origin < p, cnt_all[k, :, 0:n_exp], 0.0)
        rank_full = excl + offs
        rank_tok = jnp.sum(rank_full * onehot, axis=1, keepdims=True)
        kept = rank_tok < CAPACITY

        def block_out(w_ref, origin, row, js=tuple(range(E_PER))):
            acc = jnp.zeros((m_tok, d_ff), jnp.float32)
            for j in js:
                e = E_PER * origin + j
                s = cnt_all[row, 0, n_exp + j]
                m = jnp.where((r == e) & kept, s, 0.0)
                acc = acc + jnp.dot(
                    xb * m.astype(jnp.bfloat16),
                    w_ref[j, :, :].astype(jnp.bfloat16),
                    preferred_element_type=jnp.float32,
                )
            return acc

        out_ref[:, :] = block_out(myb, p, 0)

        h1r.wait_recv()
        h1l.wait_recv()
        h2r = wcopy(bL.at[pl.ds(0, E_PER // 2)], bD.at[pl.ds(0, E_PER // 2)],
                    2, right)
        h2l = wcopy(bR.at[pl.ds(E_PER // 2, E_PER // 2)],
                    bD.at[pl.ds(E_PER // 2, E_PER // 2)], 3, left)
        h2r.start()
        h2l.start()

        out_ref[:, :] = out_ref[:, :] + block_out(bL, left, 1)
        out_ref[:, :] = out_ref[:, :] + block_out(bR, right, 3)

        far = lax.rem(p + 2, N_DEV)
        h2r.wait_recv()
        out_ref[:, :] = out_ref[:, :] + block_out(
            bD, far, 2, js=tuple(range(E_PER // 2)))
        h2l.wait_recv()
        out_ref[:, :] = out_ref[:, :] + block_out(
            bD, far, 2, js=tuple(range(E_PER // 2, E_PER)))

        for rd in (h1r, h1l, h2r, h2l):
            rd.wait_send()

    return pl.pallas_call(
        body,
        out_shape=jax.ShapeDtypeStruct((m_tok, d_ff), jnp.float32),
        in_specs=[
            pl.BlockSpec(memory_space=pltpu.VMEM),
            pl.BlockSpec(memory_space=pltpu.VMEM),
            pl.BlockSpec(memory_space=pltpu.VMEM),
        ],
        out_specs=pl.BlockSpec(memory_space=pltpu.VMEM),
        scratch_shapes=[
            pltpu.VMEM((N_DEV, 1, n_exp + E_PER), jnp.float32),
            pltpu.VMEM((E_PER, d_model, d_ff), jnp.int8),
            pltpu.VMEM((E_PER, d_model, d_ff), jnp.int8),
            pltpu.VMEM((E_PER, d_model, d_ff), jnp.int8),
            pltpu.VMEM((E_PER, d_model, d_ff), jnp.int8),
            pltpu.SemaphoreType.DMA((N_DEV - 1,)),
            pltpu.SemaphoreType.DMA((N_DEV - 1,)),
            pltpu.SemaphoreType.DMA((4,)),
            pltpu.SemaphoreType.DMA((4,)),
        ],
        compiler_params=pltpu.CompilerParams(
            collective_id=0,
            vmem_limit_bytes=64 * 1024 * 1024,
        ),
    )(x, route_idx, expert_W)
